# baseline (speedup 1.0000x reference)
import functools

import jax
import jax.numpy as jnp
from jax import lax
from jax.experimental import pallas as pl
from jax.experimental.pallas import tpu as pltpu
from jax.experimental.pallas import tpu_sc as plsc

N = 10000
E = 320000
DF = 128
DE = 16
DG = 16
G = 100
H = 8
C = 64

NSC = 2
NTILE = 16
NW = NSC * NTILE
KF = 112
EPAD = 322560
EXTRA = 224
EPT_A = EPAD // NW
NCH_A = EPT_A // KF
EPT_B = EPAD // NTILE
NCH_B = EPT_B // KF
NP = 10112
RPT = NP // NTILE

_SC_PARAMS = pltpu.CompilerParams(use_tc_tiling_on_sc=False,
                                  needs_layout_passes=False)


def _leaky(x):
    return jnp.maximum(x, 0.2 * x)


def _make_pass_a(with_attr):
    mesh = plsc.VectorSubcoreMesh(core_axis_name="c", subcore_axis_name="s")
    out_type = [
        jax.ShapeDtypeStruct((EPAD + EXTRA, 16), jnp.float32),
        jax.ShapeDtypeStruct((NSC * NP, 16), jnp.float32),
    ]
    scratch = [
        pltpu.VMEM_SHARED((NP, 16), jnp.float32),
        pltpu.VMEM((16,), jnp.float32),
    ]
    for _ in range(2):
        scratch += [
            pltpu.VMEM((KF,), jnp.int32),
            pltpu.VMEM((KF,), jnp.int32),
            pltpu.VMEM((KF, 16), jnp.float32),
            pltpu.VMEM((KF, 16), jnp.float32),
            pltpu.VMEM((KF, 16), jnp.float32),
            pltpu.VMEM((KF, 16), jnp.float32),
            pltpu.SemaphoreType.DMA,
            pltpu.SemaphoreType.DMA,
        ]
    if with_attr:
        out_type.append(jax.ShapeDtypeStruct((NSC * NP, 16), jnp.float32))
        scratch.append(pltpu.VMEM_SHARED((NP, 16), jnp.float32))
        scratch.append(pltpu.VMEM((KF, 16), jnp.float32))
        scratch.append(pltpu.VMEM((KF, 16), jnp.float32))

    def body(s16, d16, ae16, src, dst, m16, z16, ea, n_out, den_out, *rest):
        if with_attr:
            attr_out = rest[0]
            rest = rest[1:]
        den_sh, m_v = rest[0], rest[1]
        slots = [rest[2:10], rest[10:18]]
        if with_attr:
            attr_sh = rest[18]
            eav = [rest[19], rest[20]]

        cid = lax.axis_index("c")
        sid = lax.axis_index("s")
        wid = cid * NTILE + sid
        r0 = sid * RPT

        pltpu.sync_copy(z16.at[pl.ds(r0, RPT)], den_sh.at[pl.ds(r0, RPT)])
        if with_attr:
            pltpu.sync_copy(z16.at[pl.ds(r0, RPT)], attr_sh.at[pl.ds(r0, RPT)])
        pltpu.sync_copy(m16, m_v)
        plsc.subcore_barrier()

        lane = lax.broadcasted_iota(jnp.int32, (16,), 0)
        degrow = jnp.where(lane == 8, 1.0, 0.0)
        mv = m_v[...]
        e0 = wid * EPT_A

        def prefetch(b, ci):
            idxs, idxd, sv, dv, aev, nv, semS, semD = slots[b]
            base = e0 + ci * KF
            pltpu.sync_copy(src.at[pl.ds(base, KF)], idxs)
            pltpu.sync_copy(dst.at[pl.ds(base, KF)], idxd)
            pltpu.sync_copy(ae16.at[pl.ds(base, KF)], aev)
            if with_attr:
                pltpu.sync_copy(ea.at[pl.ds(base, KF)], eav[b])
            pltpu.async_copy(s16.at[idxs], sv, semS)
            pltpu.async_copy(d16.at[idxd], dv, semD)

        def process(b, ci):
            idxs, idxd, sv, dv, aev, nv, semS, semD = slots[b]
            base = e0 + ci * KF
            pltpu.make_async_copy(s16.at[idxs], sv, semS).wait()
            pltpu.make_async_copy(d16.at[idxd], dv, semD).wait()

            def row(i, carry):
                t = sv[i, :] + dv[i, :] + aev[i, :]
                g = dv[i, :] + mv
                val = jnp.exp(_leaky(t) - _leaky(g))
                nv[i, :] = jnp.where(lane < 8, val, degrow)
                return carry

            lax.fori_loop(0, KF, row, 0, unroll=4)
            pltpu.sync_copy(nv, n_out.at[pl.ds(base, KF)])
            pltpu.sync_copy(nv, den_sh.at[idxd], add=True)
            if with_attr:
                pltpu.sync_copy(eav[b], attr_sh.at[idxd], add=True)

        prefetch(0, 0)
        prefetch(1, 1)

        def pair(ci, carry):
            for b in range(2):
                process(b, 2 * ci + b)
                prefetch(b, 2 * ci + b + 2)
            return carry

        lax.fori_loop(0, NCH_A // 2, pair, 0)
        for b in range(2):
            idxsb, idxdb, svb, dvb = slots[b][0], slots[b][1], slots[b][2], slots[b][3]
            semSb, semDb = slots[b][6], slots[b][7]
            pltpu.make_async_copy(s16.at[idxsb], svb, semSb).wait()
            pltpu.make_async_copy(d16.at[idxdb], dvb, semDb).wait()

        plsc.subcore_barrier()
        dump0 = cid * NP + sid * RPT
        pltpu.sync_copy(den_sh.at[pl.ds(sid * RPT, RPT)],
                        den_out.at[pl.ds(dump0, RPT)])
        if with_attr:
            pltpu.sync_copy(attr_sh.at[pl.ds(sid * RPT, RPT)],
                            attr_out.at[pl.ds(dump0, RPT)])

    return pl.kernel(body, out_type=out_type, mesh=mesh,
                     scratch_types=scratch, compiler_params=_SC_PARAMS)


_pass_a_attr = _make_pass_a(True)
_pass_a_plain = _make_pass_a(False)


def _pass_b_body(hp, n16, src, dst, z128, out_hbm, out_sh, *slots_flat):
    slots = [slots_flat[0:5], slots_flat[5:10]]
    idxdS, srows, semC = slots_flat[10], slots_flat[11], slots_flat[12]
    cid = lax.axis_index("c")
    sid = lax.axis_index("s")
    r0 = sid * RPT
    e0 = sid * EPT_B

    for p in range(2):
        pp = cid * 2 + p
        h0 = pp * 2
        goff = pp * N
        off = pp * NP
        pltpu.sync_copy(z128.at[pl.ds(r0, RPT)], out_sh.at[pl.ds(r0, RPT)])
        plsc.subcore_barrier()

        def prefetch(b, ci):
            idxs, idxd, nvb, rows, semG = slots[b]
            base = e0 + ci * KF
            pltpu.sync_copy(src.at[pl.ds(base, KF)], idxs)
            pltpu.sync_copy(dst.at[pl.ds(base, KF)], idxd)
            pltpu.sync_copy(n16.at[pl.ds(base, KF)], nvb)
            gv = jnp.broadcast_to(goff, (16,))
            for j in range(KF // 16):
                sl = pl.ds(j * 16, 16)
                idxs[sl] = idxs[sl] + gv
            pltpu.async_copy(hp.at[idxs], rows, semG)

        hv0 = jnp.broadcast_to(h0, (16,))
        hv1 = jnp.broadcast_to(h0 + 1, (16,))

        def process(b, ci):
            idxs, idxd, nvb, rows, semG = slots[b]
            pltpu.make_async_copy(hp.at[idxs], rows, semG).wait()
            pltpu.make_async_copy(srows, out_sh.at[idxdS], semC).wait()
            for j in range(KF // 16):
                sl = pl.ds(j * 16, 16)
                idxdS[sl] = idxd[sl]

            def scale(i, carry):
                iv = jnp.broadcast_to(i, (16,))
                s0 = plsc.load_gather(nvb, [iv, hv0])
                s1 = plsc.load_gather(nvb, [iv, hv1])
                for q in range(8):
                    sl = pl.ds(q * 16, 16)
                    srows[i, sl] = rows[i, sl] * (s0 if q < 4 else s1)
                return carry

            lax.fori_loop(0, KF, scale, 0, unroll=4)
            pltpu.async_copy(srows, out_sh.at[idxdS], semC, add=True)

        prefetch(0, 0)
        prefetch(1, 1)
        pltpu.sync_copy(z128.at[pl.ds(0, KF)], srows)
        for j in range(KF // 16):
            sl = pl.ds(j * 16, 16)
            idxdS[sl] = slots[0][1][sl]
        pltpu.async_copy(srows, out_sh.at[idxdS], semC, add=True)

        def pair_iter(ci, carry):
            for b in range(2):
                process(b, 2 * ci + b)
                prefetch(b, 2 * ci + b + 2)
            return carry

        lax.fori_loop(0, NCH_B // 2, pair_iter, 0)
        for b in range(2):
            idxs, idxd, nvb, rows, semG = slots[b]
            pltpu.make_async_copy(hp.at[idxs], rows, semG).wait()
        pltpu.make_async_copy(srows, out_sh.at[idxdS], semC).wait()

        plsc.subcore_barrier()
        pltpu.sync_copy(out_sh.at[pl.ds(r0, RPT)],
                        out_hbm.at[pl.ds(off + r0, RPT)])
        plsc.subcore_barrier()


def _pass_b_slot_scratch():
    return [
        pltpu.VMEM((KF,), jnp.int32),
        pltpu.VMEM((KF,), jnp.int32),
        pltpu.VMEM((KF, 16), jnp.float32),
        pltpu.VMEM((KF, 2 * C), jnp.float32),
        pltpu.SemaphoreType.DMA,
    ]


_pass_b = pl.kernel(
    _pass_b_body,
    out_type=[jax.ShapeDtypeStruct((4 * NP, 2 * C), jnp.float32)],
    mesh=plsc.VectorSubcoreMesh(core_axis_name="c", subcore_axis_name="s"),
    scratch_types=[pltpu.VMEM_SHARED((NP, 2 * C), jnp.float32)]
    + _pass_b_slot_scratch() + _pass_b_slot_scratch()
    + [pltpu.VMEM((KF,), jnp.int32),
       pltpu.VMEM((KF, 2 * C), jnp.float32),
       pltpu.SemaphoreType.DMA],
    compiler_params=_SC_PARAMS,
)


BN = 400
EB = 5888
GP = 104


def _dense_body(x_ref, w_ref, vs_ref, vd_ref, h_ref, hp_ref, s16_ref, d16_ref,
                mx_ref):
    i = pl.program_id(0)
    xb = x_ref[...]
    hb = xb @ w_ref[...]
    h_ref[...] = hb
    for p in range(H // 2):
        hp_ref[p] = hb[:, p * 2 * C:(p + 1) * 2 * C]
    als = xb @ vs_ref[...]
    ald = xb @ vd_ref[...]
    s16_ref[...] = jnp.concatenate([als, als], axis=1)
    d16_ref[...] = jnp.concatenate([ald, ald], axis=1)
    bm = als.max(axis=0)[None, :]

    @pl.when(i == 0)
    def _():
        mx_ref[...] = bm

    @pl.when(i > 0)
    def _():
        mx_ref[...] = jnp.maximum(mx_ref[...], bm)


def _tc_dense(x_in, W, Vs, Vd):
    dfin = x_in.shape[1]
    return pl.pallas_call(
        _dense_body,
        grid=(N // BN,),
        in_specs=[
            pl.BlockSpec((BN, dfin), lambda i: (i, 0)),
            pl.BlockSpec((dfin, H * C), lambda i: (0, 0)),
            pl.BlockSpec((dfin, H), lambda i: (0, 0)),
            pl.BlockSpec((dfin, H), lambda i: (0, 0)),
        ],
        out_specs=[
            pl.BlockSpec((BN, H * C), lambda i: (i, 0)),
            pl.BlockSpec((H // 2, BN, 2 * C), lambda i: (0, i, 0)),
            pl.BlockSpec((BN, 16), lambda i: (i, 0)),
            pl.BlockSpec((BN, 16), lambda i: (i, 0)),
            pl.BlockSpec((1, H), lambda i: (0, 0)),
        ],
        out_shape=[
            jax.ShapeDtypeStruct((N, H * C), jnp.float32),
            jax.ShapeDtypeStruct((H // 2, N, 2 * C), jnp.float32),
            jax.ShapeDtypeStruct((N, 16), jnp.float32),
            jax.ShapeDtypeStruct((N, 16), jnp.float32),
            jax.ShapeDtypeStruct((1, H), jnp.float32),
        ],
    )(x_in, W, Vs, Vd)


def _elog_body(ae_ref, ve_ref, ae16_ref, mx_ref):
    i = pl.program_id(0)
    alb = ae_ref[...] @ ve_ref[...]
    ae16_ref[...] = jnp.concatenate([alb, alb], axis=1)
    bm = alb.max(axis=0)[None, :]

    @pl.when(i == 0)
    def _():
        mx_ref[...] = bm

    @pl.when(i > 0)
    def _():
        mx_ref[...] = jnp.maximum(mx_ref[...], bm)


def _tc_elog(ae_pad, Ve):
    return pl.pallas_call(
        _elog_body,
        grid=((EPAD + EXTRA) // EB,),
        in_specs=[
            pl.BlockSpec((EB, DE), lambda i: (i, 0)),
            pl.BlockSpec((DE, H), lambda i: (0, 0)),
        ],
        out_specs=[
            pl.BlockSpec((EB, 16), lambda i: (i, 0)),
            pl.BlockSpec((1, H), lambda i: (0, 0)),
        ],
        out_shape=[
            jax.ShapeDtypeStruct((EPAD + EXTRA, 16), jnp.float32),
            jax.ShapeDtypeStruct((1, H), jnp.float32),
        ],
    )(ae_pad, Ve)


def _pool_mlp_body(h_ref, bv_ref, bs_ref, u_ref, w1_ref, b1_ref, w2_ref,
                   b2_ref, w3_ref, b3_ref, out_ref):
    gi = lax.broadcasted_iota(jnp.int32, (GP, N), 0)
    onehot = jnp.where(gi == bv_ref[...], 1.0, 0.0)
    cnt = onehot.sum(axis=1)[:, None]
    ge = (onehot @ h_ref[...]) / jnp.clip(cnt, 1.0, None)
    gs = lax.broadcasted_iota(jnp.int32, (GP, G), 1)
    selhot = jnp.where(gs == bs_ref[...], 1.0, 0.0)
    usel = selhot @ u_ref[...]
    comb = jnp.concatenate([ge, usel], axis=1)
    z = jnp.maximum(comb @ w1_ref[...] + b1_ref[...], 0.0)
    z = jnp.maximum(z @ w2_ref[...] + b2_ref[...], 0.0)
    res = z @ w3_ref[...] + b3_ref[...]
    out_ref[...] = res[:G]


def _gat_layer(x_in, srcp, dstp, ae_pad, loop_attr, W, att_src, att_dst, We,
               att_edge, bias, concat, z16, z128, with_attr, edge_attr_p):
    Vs = jnp.einsum("fhc,hc->fh", W.reshape(-1, H, C), att_src[0])
    Vd = jnp.einsum("fhc,hc->fh", W.reshape(-1, H, C), att_dst[0])
    Ve = jnp.einsum("dhc,hc->dh", We.reshape(DE, H, C), att_edge[0])

    h, hp3, s16, d16n, alsmax = _tc_dense(x_in, W, Vs, Vd)
    ae16, aemax = _tc_elog(ae_pad, Ve)
    al_s, al_d = s16[:, :H], d16n[:, :H]
    M = alsmax[0] + jnp.maximum(aemax[0], 0.0)
    d16 = jnp.pad(d16n, ((0, NP - N), (0, 0)))
    m16 = jnp.concatenate([M, M])

    if with_attr:
        n16, den2, attr2 = _pass_a_attr(s16, d16, ae16, srcp, dstp, m16, z16,
                                        edge_attr_p)
        attr_sum = attr2.reshape(NSC, NP, 16)[:, :N].sum(0)
    else:
        n16, den2 = _pass_a_plain(s16, d16, ae16, srcp, dstp, m16, z16,
                                  edge_attr_p)
        attr_sum = None
    den = den2.reshape(NSC, NP, 16)[:, :N].sum(0)
    denom, deg = den[:, :H], den[:, H]

    hp = hp3.reshape((H // 2) * N, 2 * C)
    (outp,) = _pass_b(hp, n16, srcp, dstp, z128)
    outscat = outp.reshape(H // 2, NP, 2, C)[:, :N].transpose(
        1, 0, 2, 3).reshape(N, H, C)

    if loop_attr is None:
        loop_attr = attr_sum / jnp.clip(deg, 1.0, None)[:, None]
    al_e_loop = loop_attr @ Ve
    stab = _leaky(al_d + M[None, :])
    n_self = jnp.exp(_leaky(al_s + al_d + al_e_loop) - stab)

    h3 = h.reshape(N, H, C)
    out = ((outscat + h3 * n_self[:, :, None])
           / (denom + n_self)[:, :, None])
    if concat:
        out = out.reshape(N, H * C) + bias
    else:
        out = out.mean(axis=1) + bias
    return jax.nn.elu(out), loop_attr


def kernel(x, edge_index, edge_attr, u, batch, W1, att_src1, att_dst1, We1,
           att_edge1, b1, W2, att_src2, att_dst2, We2, att_edge2, b2, m1w,
           m1b, m2w, m2b, m3w, m3b):
    src, dst = edge_index[0], edge_index[1]
    npad = EPAD + EXTRA - E
    srcp = jnp.concatenate([src, jnp.zeros((npad,), src.dtype)])
    dstp = jnp.concatenate([dst, jnp.full((npad,), NP - 1, dst.dtype)])
    ae_pad = jnp.pad(edge_attr, ((0, npad), (0, 0)))
    z16 = jnp.zeros((NP, 16), jnp.float32)
    z128 = jnp.zeros((NP, 2 * C), jnp.float32)

    h, loop_attr = _gat_layer(x, srcp, dstp, ae_pad, None, W1, att_src1,
                              att_dst1, We1, att_edge1, b1, True, z16, z128,
                              True, ae_pad)
    h, _ = _gat_layer(h, srcp, dstp, ae_pad, loop_attr, W2, att_src2,
                      att_dst2, We2, att_edge2, b2, False, z16, z128,
                      False, ae_pad)

    stride = N // G
    out = pl.pallas_call(
        _pool_mlp_body,
        out_shape=jax.ShapeDtypeStruct((G, 1), jnp.float32),
    )(h, batch.reshape(1, N),
      jnp.pad(batch[::stride], (0, GP - G)).reshape(GP, 1), u, m1w,
      m1b.reshape(1, -1), m2w, m2b.reshape(1, -1), m3w, m3b.reshape(1, -1))
    return out


# speedup vs baseline: 1.3544x; 1.3544x over previous
import functools

import jax
import jax.numpy as jnp
from jax import lax
from jax.experimental import pallas as pl
from jax.experimental.pallas import tpu as pltpu
from jax.experimental.pallas import tpu_sc as plsc

N = 10000
E = 320000
DF = 128
DE = 16
DG = 16
G = 100
H = 8
C = 64

NSC = 2
NTILE = 16
NW = NSC * NTILE
KF = 128
EPAD = 323584
EXTRA = 256
EPT_A = EPAD // NW
NCH_A = EPT_A // KF
EPT_B = EPAD // NTILE
NCH_B = EPT_B // KF
NP = 10240
RPT = NP // NTILE

_SC_PARAMS = pltpu.CompilerParams(use_tc_tiling_on_sc=False,
                                  needs_layout_passes=False)


def _leaky(x):
    return jnp.maximum(x, 0.2 * x)


def _make_pass_a(with_attr):
    mesh = plsc.VectorSubcoreMesh(core_axis_name="c", subcore_axis_name="s")
    out_type = [
        jax.ShapeDtypeStruct((EPAD + EXTRA, 16), jnp.float32),
        jax.ShapeDtypeStruct((NSC * NP, 16), jnp.float32),
    ]
    scratch = [
        pltpu.VMEM_SHARED((NP, 16), jnp.float32),
        pltpu.VMEM((16,), jnp.float32),
    ]
    for _ in range(2):
        scratch += [
            pltpu.VMEM((KF,), jnp.int32),
            pltpu.VMEM((KF,), jnp.int32),
            pltpu.VMEM((KF, 16), jnp.float32),
            pltpu.VMEM((KF, 16), jnp.float32),
            pltpu.VMEM((KF, 16), jnp.float32),
            pltpu.VMEM((KF, 16), jnp.float32),
            pltpu.SemaphoreType.DMA,
            pltpu.SemaphoreType.DMA,
        ]
    if with_attr:
        out_type.append(jax.ShapeDtypeStruct((NSC * NP, 16), jnp.float32))
        scratch.append(pltpu.VMEM_SHARED((NP, 16), jnp.float32))
        scratch.append(pltpu.VMEM((KF, 16), jnp.float32))
        scratch.append(pltpu.VMEM((KF, 16), jnp.float32))

    def body(s16, d16, ae16, src, dst, m16, z16, ea, n_out, den_out, *rest):
        if with_attr:
            attr_out = rest[0]
            rest = rest[1:]
        den_sh, m_v = rest[0], rest[1]
        slots = [rest[2:10], rest[10:18]]
        if with_attr:
            attr_sh = rest[18]
            eav = [rest[19], rest[20]]

        cid = lax.axis_index("c")
        sid = lax.axis_index("s")
        wid = cid * NTILE + sid
        r0 = sid * RPT

        pltpu.sync_copy(z16.at[pl.ds(r0, RPT)], den_sh.at[pl.ds(r0, RPT)])
        if with_attr:
            pltpu.sync_copy(z16.at[pl.ds(r0, RPT)], attr_sh.at[pl.ds(r0, RPT)])
        pltpu.sync_copy(m16, m_v)
        plsc.subcore_barrier()

        lane = lax.broadcasted_iota(jnp.int32, (16,), 0)
        degrow = jnp.where(lane == 8, 1.0, 0.0)
        mv = m_v[...]
        e0 = wid * EPT_A

        def prefetch(b, ci):
            idxs, idxd, sv, dv, aev, nv, semS, semD = slots[b]
            base = e0 + ci * KF
            pltpu.sync_copy(src.at[pl.ds(base, KF)], idxs)
            pltpu.sync_copy(dst.at[pl.ds(base, KF)], idxd)
            pltpu.sync_copy(ae16.at[pl.ds(base, KF)], aev)
            if with_attr:
                pltpu.sync_copy(ea.at[pl.ds(base, KF)], eav[b])
            pltpu.async_copy(s16.at[idxs], sv, semS)
            pltpu.async_copy(d16.at[idxd], dv, semD)

        def process(b, ci):
            idxs, idxd, sv, dv, aev, nv, semS, semD = slots[b]
            base = e0 + ci * KF
            pltpu.make_async_copy(s16.at[idxs], sv, semS).wait()
            pltpu.make_async_copy(d16.at[idxd], dv, semD).wait()

            def row(i, carry):
                t = sv[i, :] + dv[i, :] + aev[i, :]
                g = dv[i, :] + mv
                val = jnp.exp(_leaky(t) - _leaky(g))
                nv[i, :] = jnp.where(lane < 8, val, degrow)
                return carry

            lax.fori_loop(0, KF, row, 0, unroll=8)
            pltpu.sync_copy(nv, n_out.at[pl.ds(base, KF)])
            pltpu.sync_copy(nv, den_sh.at[idxd], add=True)
            if with_attr:
                pltpu.sync_copy(eav[b], attr_sh.at[idxd], add=True)

        prefetch(0, 0)
        prefetch(1, 1)

        def pair(ci, carry):
            for b in range(2):
                process(b, 2 * ci + b)
                prefetch(b, 2 * ci + b + 2)
            return carry

        lax.fori_loop(0, (NCH_A - 1) // 2, pair, 0)
        process(0, NCH_A - 1)
        _, _, sv1, dv1, _, _, semS1, semD1 = slots[1]
        idxs1, idxd1 = slots[1][0], slots[1][1]
        pltpu.make_async_copy(s16.at[idxs1], sv1, semS1).wait()
        pltpu.make_async_copy(d16.at[idxd1], dv1, semD1).wait()

        plsc.subcore_barrier()
        dump0 = cid * NP + sid * RPT
        pltpu.sync_copy(den_sh.at[pl.ds(sid * RPT, RPT)],
                        den_out.at[pl.ds(dump0, RPT)])
        if with_attr:
            pltpu.sync_copy(attr_sh.at[pl.ds(sid * RPT, RPT)],
                            attr_out.at[pl.ds(dump0, RPT)])

    return pl.kernel(body, out_type=out_type, mesh=mesh,
                     scratch_types=scratch, compiler_params=_SC_PARAMS)


_pass_a_attr = _make_pass_a(True)
_pass_a_plain = _make_pass_a(False)


def _pass_b_body(hp, n16, src, dst, z128, out_hbm, out_sh, *slots_flat):
    slots = [slots_flat[0:6], slots_flat[6:12]]
    cid = lax.axis_index("c")
    sid = lax.axis_index("s")
    r0 = sid * RPT
    e0 = sid * EPT_B

    for p in range(2):
        pp = cid * 2 + p
        h0 = pp * 2
        goff = pp * N
        off = pp * NP
        pltpu.sync_copy(z128.at[pl.ds(r0, RPT)], out_sh.at[pl.ds(r0, RPT)])
        plsc.subcore_barrier()

        def prefetch(b, ci):
            idxs, idxd, nvb, rows, semG, _ = slots[b]
            base = e0 + ci * KF
            pltpu.sync_copy(src.at[pl.ds(base, KF)], idxs)
            pltpu.sync_copy(dst.at[pl.ds(base, KF)], idxd)
            pltpu.sync_copy(n16.at[pl.ds(base, KF)], nvb)
            gv = jnp.broadcast_to(goff, (16,))
            for j in range(KF // 16):
                sl = pl.ds(j * 16, 16)
                idxs[sl] = idxs[sl] + gv
            pltpu.async_copy(hp.at[idxs], rows, semG)

        hv0 = jnp.broadcast_to(h0, (16,))
        hv1 = jnp.broadcast_to(h0 + 1, (16,))

        def process(b, ci):
            idxs, idxd, nvb, rows, semG, _ = slots[b]
            pltpu.make_async_copy(hp.at[idxs], rows, semG).wait()

            def scale(i, carry):
                iv = jnp.broadcast_to(i, (16,))
                s0 = plsc.load_gather(nvb, [iv, hv0])
                s1 = plsc.load_gather(nvb, [iv, hv1])
                for q in range(8):
                    sl = pl.ds(q * 16, 16)
                    rows[i, sl] = rows[i, sl] * (s0 if q < 4 else s1)
                return carry

            lax.fori_loop(0, KF, scale, 0, unroll=8)
            pltpu.sync_copy(rows, out_sh.at[idxd], add=True)

        prefetch(0, 0)
        prefetch(1, 1)

        def pair_iter(ci, carry):
            for b in range(2):
                process(b, 2 * ci + b)
                prefetch(b, 2 * ci + b + 2)
            return carry

        lax.fori_loop(0, NCH_B // 2, pair_iter, 0)
        for b in range(2):
            idxs, idxd, nvb, rows, semG, _ = slots[b]
            pltpu.make_async_copy(hp.at[idxs], rows, semG).wait()

        plsc.subcore_barrier()
        pltpu.sync_copy(out_sh.at[pl.ds(r0, RPT)],
                        out_hbm.at[pl.ds(off + r0, RPT)])
        plsc.subcore_barrier()


def _pass_b_slot_scratch():
    return [
        pltpu.VMEM((KF,), jnp.int32),
        pltpu.VMEM((KF,), jnp.int32),
        pltpu.VMEM((KF, 16), jnp.float32),
        pltpu.VMEM((KF, 2 * C), jnp.float32),
        pltpu.SemaphoreType.DMA,
        pltpu.SemaphoreType.DMA,
    ]


_pass_b = pl.kernel(
    _pass_b_body,
    out_type=[jax.ShapeDtypeStruct((4 * NP, 2 * C), jnp.float32)],
    mesh=plsc.VectorSubcoreMesh(core_axis_name="c", subcore_axis_name="s"),
    scratch_types=[pltpu.VMEM_SHARED((NP, 2 * C), jnp.float32)]
    + _pass_b_slot_scratch() + _pass_b_slot_scratch(),
    compiler_params=_SC_PARAMS,
)


BN = 400
EB = 5888
GP = 104


def _dense_body(x_ref, w_ref, vs_ref, vd_ref, h_ref, hp_ref, s16_ref, d16_ref,
                mx_ref):
    i = pl.program_id(0)
    xb = x_ref[...]
    hb = xb @ w_ref[...]
    h_ref[...] = hb
    for p in range(H // 2):
        hp_ref[p] = hb[:, p * 2 * C:(p + 1) * 2 * C]
    als = xb @ vs_ref[...]
    ald = xb @ vd_ref[...]
    s16_ref[...] = jnp.concatenate([als, als], axis=1)
    d16_ref[...] = jnp.concatenate([ald, ald], axis=1)
    bm = als.max(axis=0)[None, :]

    @pl.when(i == 0)
    def _():
        mx_ref[...] = bm

    @pl.when(i > 0)
    def _():
        mx_ref[...] = jnp.maximum(mx_ref[...], bm)


def _tc_dense(x_in, W, Vs, Vd):
    dfin = x_in.shape[1]
    return pl.pallas_call(
        _dense_body,
        grid=(N // BN,),
        in_specs=[
            pl.BlockSpec((BN, dfin), lambda i: (i, 0)),
            pl.BlockSpec((dfin, H * C), lambda i: (0, 0)),
            pl.BlockSpec((dfin, H), lambda i: (0, 0)),
            pl.BlockSpec((dfin, H), lambda i: (0, 0)),
        ],
        out_specs=[
            pl.BlockSpec((BN, H * C), lambda i: (i, 0)),
            pl.BlockSpec((H // 2, BN, 2 * C), lambda i: (0, i, 0)),
            pl.BlockSpec((BN, 16), lambda i: (i, 0)),
            pl.BlockSpec((BN, 16), lambda i: (i, 0)),
            pl.BlockSpec((1, H), lambda i: (0, 0)),
        ],
        out_shape=[
            jax.ShapeDtypeStruct((N, H * C), jnp.float32),
            jax.ShapeDtypeStruct((H // 2, N, 2 * C), jnp.float32),
            jax.ShapeDtypeStruct((N, 16), jnp.float32),
            jax.ShapeDtypeStruct((N, 16), jnp.float32),
            jax.ShapeDtypeStruct((1, H), jnp.float32),
        ],
    )(x_in, W, Vs, Vd)


def _elog_body(ae_ref, ve_ref, ae16_ref, mx_ref):
    i = pl.program_id(0)
    alb = ae_ref[...] @ ve_ref[...]
    ae16_ref[...] = jnp.concatenate([alb, alb], axis=1)
    bm = alb.max(axis=0)[None, :]

    @pl.when(i == 0)
    def _():
        mx_ref[...] = bm

    @pl.when(i > 0)
    def _():
        mx_ref[...] = jnp.maximum(mx_ref[...], bm)


def _tc_elog(ae_pad, Ve):
    return pl.pallas_call(
        _elog_body,
        grid=((EPAD + EXTRA) // EB,),
        in_specs=[
            pl.BlockSpec((EB, DE), lambda i: (i, 0)),
            pl.BlockSpec((DE, H), lambda i: (0, 0)),
        ],
        out_specs=[
            pl.BlockSpec((EB, 16), lambda i: (i, 0)),
            pl.BlockSpec((1, H), lambda i: (0, 0)),
        ],
        out_shape=[
            jax.ShapeDtypeStruct((EPAD + EXTRA, 16), jnp.float32),
            jax.ShapeDtypeStruct((1, H), jnp.float32),
        ],
    )(ae_pad, Ve)


def _pool_mlp_body(h_ref, bv_ref, bs_ref, u_ref, w1_ref, b1_ref, w2_ref,
                   b2_ref, w3_ref, b3_ref, out_ref):
    gi = lax.broadcasted_iota(jnp.int32, (GP, N), 0)
    onehot = jnp.where(gi == bv_ref[...], 1.0, 0.0)
    cnt = onehot.sum(axis=1)[:, None]
    ge = (onehot @ h_ref[...]) / jnp.clip(cnt, 1.0, None)
    gs = lax.broadcasted_iota(jnp.int32, (GP, G), 1)
    selhot = jnp.where(gs == bs_ref[...], 1.0, 0.0)
    usel = selhot @ u_ref[...]
    comb = jnp.concatenate([ge, usel], axis=1)
    z = jnp.maximum(comb @ w1_ref[...] + b1_ref[...], 0.0)
    z = jnp.maximum(z @ w2_ref[...] + b2_ref[...], 0.0)
    res = z @ w3_ref[...] + b3_ref[...]
    out_ref[...] = res[:G]


def _gat_layer(x_in, srcp, dstp, ae_pad, loop_attr, W, att_src, att_dst, We,
               att_edge, bias, concat, z16, z128, with_attr, edge_attr_p):
    Vs = jnp.einsum("fhc,hc->fh", W.reshape(-1, H, C), att_src[0])
    Vd = jnp.einsum("fhc,hc->fh", W.reshape(-1, H, C), att_dst[0])
    Ve = jnp.einsum("dhc,hc->dh", We.reshape(DE, H, C), att_edge[0])

    h, hp3, s16, d16n, alsmax = _tc_dense(x_in, W, Vs, Vd)
    ae16, aemax = _tc_elog(ae_pad, Ve)
    al_s, al_d = s16[:, :H], d16n[:, :H]
    M = alsmax[0] + jnp.maximum(aemax[0], 0.0)
    d16 = jnp.pad(d16n, ((0, NP - N), (0, 0)))
    m16 = jnp.concatenate([M, M])

    if with_attr:
        n16, den2, attr2 = _pass_a_attr(s16, d16, ae16, srcp, dstp, m16, z16,
                                        edge_attr_p)
        attr_sum = attr2.reshape(NSC, NP, 16)[:, :N].sum(0)
    else:
        n16, den2 = _pass_a_plain(s16, d16, ae16, srcp, dstp, m16, z16,
                                  edge_attr_p)
        attr_sum = None
    den = den2.reshape(NSC, NP, 16)[:, :N].sum(0)
    denom, deg = den[:, :H], den[:, H]

    hp = hp3.reshape((H // 2) * N, 2 * C)
    (outp,) = _pass_b(hp, n16, srcp, dstp, z128)
    outscat = outp.reshape(H // 2, NP, 2, C)[:, :N].transpose(
        1, 0, 2, 3).reshape(N, H, C)

    if loop_attr is None:
        loop_attr = attr_sum / jnp.clip(deg, 1.0, None)[:, None]
    al_e_loop = loop_attr @ Ve
    stab = _leaky(al_d + M[None, :])
    n_self = jnp.exp(_leaky(al_s + al_d + al_e_loop) - stab)

    h3 = h.reshape(N, H, C)
    out = ((outscat + h3 * n_self[:, :, None])
           / (denom + n_self)[:, :, None])
    if concat:
        out = out.reshape(N, H * C) + bias
    else:
        out = out.mean(axis=1) + bias
    return jax.nn.elu(out), loop_attr


def kernel(x, edge_index, edge_attr, u, batch, W1, att_src1, att_dst1, We1,
           att_edge1, b1, W2, att_src2, att_dst2, We2, att_edge2, b2, m1w,
           m1b, m2w, m2b, m3w, m3b):
    src, dst = edge_index[0], edge_index[1]
    npad = EPAD + EXTRA - E
    srcp = jnp.concatenate([src, jnp.zeros((npad,), src.dtype)])
    dstp = jnp.concatenate([dst, jnp.full((npad,), NP - 1, dst.dtype)])
    ae_pad = jnp.pad(edge_attr, ((0, npad), (0, 0)))
    z16 = jnp.zeros((NP, 16), jnp.float32)
    z128 = jnp.zeros((NP, 2 * C), jnp.float32)

    h, loop_attr = _gat_layer(x, srcp, dstp, ae_pad, None, W1, att_src1,
                              att_dst1, We1, att_edge1, b1, True, z16, z128,
                              True, ae_pad)
    h, _ = _gat_layer(h, srcp, dstp, ae_pad, loop_attr, W2, att_src2,
                      att_dst2, We2, att_edge2, b2, False, z16, z128,
                      False, ae_pad)

    stride = N // G
    out = pl.pallas_call(
        _pool_mlp_body,
        out_shape=jax.ShapeDtypeStruct((G, 1), jnp.float32),
    )(h, batch.reshape(1, N),
      jnp.pad(batch[::stride], (0, GP - G)).reshape(GP, 1), u, m1w,
      m1b.reshape(1, -1), m2w, m2b.reshape(1, -1), m3w, m3b.reshape(1, -1))
    return out
